# baseline (speedup 1.0000x reference)
import functools

import jax
import jax.numpy as jnp
from jax import lax
from jax.experimental import pallas as pl
from jax.experimental.pallas import tpu as pltpu
from jax.experimental.pallas import tpu_sc as plsc

_H = 128
_NF = 50000
_NU = 10000
_N_SMALL = {"worked_at": 10, "in": 150, "had": 50}

_E = 200000
_EP = 212992
_ROWS = _EP // 128
_RPT = _ROWS // 16
_G = 8
_NI = _RPT // _G
_NUP = 10112
_NFP = 50048


def _mm_kernel(x_ref, w_ref, o_ref, acc_ref, *, nk):
    @pl.when(pl.program_id(2) == 0)
    def _init():
        acc_ref[...] = jnp.zeros_like(acc_ref)

    acc_ref[...] += jnp.dot(x_ref[...], w_ref[...],
                            preferred_element_type=jnp.float32)

    @pl.when(pl.program_id(2) == nk - 1)
    def _fin():
        o_ref[...] = acc_ref[...]


def _ceil_to(x, m):
    return -(-x // m) * m


def _mm(x, w, bm, bn, bk):
    m, k = x.shape
    _, n = w.shape
    mp, kp, np_ = _ceil_to(m, bm), _ceil_to(k, bk), _ceil_to(n, bn)
    if mp > m or kp > k:
        x = jnp.pad(x, ((0, mp - m), (0, kp - k)))
    if kp > k or np_ > n:
        w = jnp.pad(w, ((0, kp - k), (0, np_ - n)))
    nk = kp // bk
    out = pl.pallas_call(
        functools.partial(_mm_kernel, nk=nk),
        grid=(mp // bm, np_ // bn, nk),
        in_specs=[
            pl.BlockSpec((bm, bk), lambda i, j, kk: (i, kk)),
            pl.BlockSpec((bk, bn), lambda i, j, kk: (kk, j)),
        ],
        out_specs=pl.BlockSpec((bm, bn), lambda i, j, kk: (i, j)),
        out_shape=jax.ShapeDtypeStruct((mp, np_), jnp.float32),
        scratch_shapes=[pltpu.VMEM((bm, bn), jnp.float32)],
        compiler_params=pltpu.CompilerParams(
            dimension_semantics=("parallel", "parallel", "arbitrary")),
    )(x, w)
    if mp > m or np_ > n:
        out = out[:m, :n]
    return out


def _mm_big(x, w):
    return _mm(x, w, bm=1024, bn=128, bk=_ceil_to(x.shape[1], 128))


def _sc_pass(s, gtab, gidx, sidx, out, n_out, acc, gbuf, sbuf, rows, sem,
             zeros_hbm):
    zr = n_out // 16
    pltpu.sync_copy(zeros_hbm.at[pl.ds(0, zr)], acc.at[pl.ds(s * zr, zr)])
    plsc.subcore_barrier()

    row0 = s * _RPT

    def body(j, carry):
        base = row0 + j * _G
        pltpu.sync_copy(gidx.at[pl.ds(base, _G)], gbuf)
        pltpu.sync_copy(sidx.at[pl.ds(base, _G)], sbuf)
        for half in range(2):
            cps = [pltpu.async_copy(
                gtab.at[gbuf.at[half * 4 + jj]], rows.at[jj], sem)
                for jj in range(4)]
            for cp in cps:
                cp.wait()
            for jj in range(4):
                pltpu.sync_copy(rows.at[jj], acc.at[sbuf.at[half * 4 + jj]],
                                add=True)
        return carry

    lax.fori_loop(0, _NI, body, 0)
    plsc.subcore_barrier()
    dr = n_out // 16
    pltpu.sync_copy(acc.at[pl.ds(s * dr, dr)], out.at[pl.ds(s * dr, dr)])
    plsc.subcore_barrier()


def _seg_body(hf0, hf1, hf2, hf3, hu0, hu1, hu2, hu3,
              gidx_f, sidx_f, gidx_r, sidx_r, zeros_hbm,
              ou0, ou1, ou2, ou3, of0, of1, of2, of3,
              acc, gbuf, sbuf, rows, sem):
    c = lax.axis_index("c")
    s = lax.axis_index("s")
    hf = (hf0, hf1, hf2, hf3)
    hu = (hu0, hu1, hu2, hu3)
    ou = (ou0, ou1, ou2, ou3)
    of = (of0, of1, of2, of3)
    for cv in (0, 1):
        @pl.when(c == cv)
        def _(cv=cv):
            for q in (2 * cv, 2 * cv + 1):
                _sc_pass(s, hf[q], gidx_f, sidx_f, ou[q], _NUP,
                         acc, gbuf, sbuf, rows, sem, zeros_hbm)
                _sc_pass(s, hu[q], gidx_r, sidx_r, of[q], _NFP,
                         acc, gbuf, sbuf, rows, sem, zeros_hbm)


def _make_seg_call():
    mesh = plsc.VectorSubcoreMesh(core_axis_name="c", subcore_axis_name="s")
    q_u = jax.ShapeDtypeStruct((_NUP, 32), jnp.float32)
    q_f = jax.ShapeDtypeStruct((_NFP, 32), jnp.float32)
    return pl.kernel(
        _seg_body,
        out_type=[q_u] * 4 + [q_f] * 4,
        mesh=mesh,
        scratch_types=[
            pltpu.VMEM_SHARED((_NFP, 32), jnp.float32),
            pltpu.VMEM((_G, 128), jnp.int32),
            pltpu.VMEM((_G, 128), jnp.int32),
            pltpu.VMEM((4, 128, 32), jnp.float32),
            pltpu.SemaphoreType.DMA,
        ],
        compiler_params=pltpu.CompilerParams(use_tc_tiling_on_sc=False),
    )


def _deg_pass(s, sidx, out, n_out, acc, sbuf, ones, sem, zeros_hbm):
    zr = n_out // 16
    pltpu.sync_copy(zeros_hbm.at[pl.ds(0, zr)], acc.at[pl.ds(s * zr, zr)])
    plsc.subcore_barrier()
    row0 = s * _RPT

    def body(j, carry):
        base = row0 + j * _G
        pltpu.sync_copy(sidx.at[pl.ds(base, _G)], sbuf)
        for jj in range(_G):
            pltpu.sync_copy(ones, acc.at[sbuf.at[jj]], add=True)
        return carry

    lax.fori_loop(0, _NI, body, 0)
    plsc.subcore_barrier()
    dr = n_out // 16
    pltpu.sync_copy(acc.at[pl.ds(s * dr, dr)], out.at[pl.ds(s * dr, dr)])


def _deg_body(sidx_f, sidx_r, zeros_hbm, ones_hbm, deg_u, deg_f,
              acc, sbuf, ones, sem):
    c = lax.axis_index("c")
    s = lax.axis_index("s")
    pltpu.sync_copy(ones_hbm, ones)

    @pl.when(c == 0)
    def _u():
        _deg_pass(s, sidx_f, deg_u, _NUP, acc, sbuf, ones, sem, zeros_hbm)

    @pl.when(c == 1)
    def _f():
        _deg_pass(s, sidx_r, deg_f, _NFP, acc, sbuf, ones, sem, zeros_hbm)


def _make_deg_call():
    mesh = plsc.VectorSubcoreMesh(core_axis_name="c", subcore_axis_name="s")
    return pl.kernel(
        _deg_body,
        out_type=[jax.ShapeDtypeStruct((_NUP, 16), jnp.float32),
                  jax.ShapeDtypeStruct((_NFP, 16), jnp.float32)],
        mesh=mesh,
        scratch_types=[
            pltpu.VMEM_SHARED((_NFP, 16), jnp.float32),
            pltpu.VMEM((_G, 128), jnp.int32),
            pltpu.VMEM((128, 16), jnp.float32),
            pltpu.SemaphoreType.DMA,
        ],
        compiler_params=pltpu.CompilerParams(use_tc_tiling_on_sc=False),
    )


def _pad_idx(idx, fill):
    return jnp.concatenate(
        [idx, jnp.full((_EP - _E,), fill, jnp.int32)]).reshape(_ROWS, 128)


def kernel(params, x_founder, x_university, x_company_size, x_industry,
           x_role_type, src_studied_at, dst_studied_at, src_worked_at,
           dst_worked_at, src_in, dst_in, src_had, dst_had):
    xs = {"founder": x_founder, "university": x_university,
          "company_size": x_company_size, "industry": x_industry,
          "role_type": x_role_type}
    rels = {"studied_at": ("university", src_studied_at, dst_studied_at),
            "worked_at": ("company_size", src_worked_at, dst_worked_at),
            "in": ("industry", src_in, dst_in),
            "had": ("role_type", src_had, dst_had)}

    proj = params["proj"]
    h = {}
    for nt, x in xs.items():
        bm = 1024 if x.shape[0] > 1024 else _ceil_to(x.shape[0], 8)
        h[nt] = _mm(x, proj[nt]["W"], bm=bm, bn=128,
                    bk=_ceil_to(x.shape[1], 128)) + proj[nt]["b"]

    M = {}
    Mt = {}
    for r, (nt, src, dst) in rels.items():
        if nt == "university":
            continue
        ns = _N_SMALL[r]
        M[r] = jnp.zeros((_NF, ns), jnp.float32).at[src, dst].add(1.0)
        Mt[r] = jnp.zeros((ns, _NF), jnp.float32).at[dst, src].add(1.0)

    gidx_f = _pad_idx(src_studied_at, 0)
    sidx_f = _pad_idx(dst_studied_at, _NU)
    gidx_r = _pad_idx(dst_studied_at, 0)
    sidx_r = _pad_idx(src_studied_at, _NF)
    zeros_hbm = jnp.zeros((3136, 32), jnp.float32)
    zeros16_hbm = jnp.zeros((3136, 16), jnp.float32)
    ones_hbm = jnp.ones((128, 16), jnp.float32)
    deg_call = _make_deg_call()
    deg_u16, deg_f16 = deg_call(sidx_f, sidx_r, zeros16_hbm, ones_hbm)
    deg_u = jnp.clip(deg_u16[:_NU, 0], 1.0, None)[:, None]
    deg_fu = jnp.clip(deg_f16[:_NF, 0], 1.0, None)[:, None]
    seg_call = _make_seg_call()

    ones_f = jnp.ones((_NF, 1), jnp.float32)

    for layer in params["convs"]:
        h_f_aug = jnp.concatenate([h["founder"], ones_f], axis=1)
        new_h = {}
        founder_acc = []

        hfq = [h["founder"][:, q * 32:(q + 1) * 32] for q in range(4)]
        huq = [h["university"][:, q * 32:(q + 1) * 32] for q in range(4)]
        outs = seg_call(*hfq, *huq, gidx_f, sidx_f, gidx_r, sidx_r, zeros_hbm)
        sum_u = jnp.concatenate([o[:_NU] for o in outs[:4]], axis=1)
        rev_sum = jnp.concatenate([o[:_NF] for o in outs[4:]], axis=1)

        for r, (nt, src, dst) in rels.items():
            p_fwd = layer["founder__" + r + "__" + nt]
            p_rev = layer[nt + "__rev_" + r + "__founder"]
            if nt == "university":
                aggr_d = sum_u / deg_u
                aggr_f = rev_sum / deg_fu
            else:
                ns = _N_SMALL[r]
                s = _mm(Mt[r], h_f_aug, bm=_ceil_to(ns, 8), bn=128, bk=1024)
                aggr_d = s[:, :_H] / jnp.clip(s[:, _H:_H + 1], 1.0, None)
                h_s_aug = jnp.concatenate(
                    [h[nt], jnp.ones((ns, 1), jnp.float32)], axis=1)
                rsum = _mm(M[r], h_s_aug, bm=1024, bn=128,
                           bk=_ceil_to(ns + 1, 128))
                aggr_f = rsum[:, :_H] / jnp.clip(rsum[:, _H:_H + 1], 1.0,
                                                 None)

            bm_d = 1024 if aggr_d.shape[0] > 1024 else _ceil_to(
                aggr_d.shape[0], 8)
            out_d = (_mm(aggr_d, p_fwd["W_l"], bm=bm_d, bn=128, bk=128)
                     + p_fwd["b_l"]
                     + _mm(h[nt], p_fwd["W_r"], bm=bm_d, bn=128, bk=128))
            new_h[nt] = jax.nn.relu(out_d)
            founder_acc.append((aggr_f, p_rev))

        w_r_mean = sum(p["W_r"] for _, p in founder_acc) * 0.25
        b_l_mean = sum(p["b_l"] for _, p in founder_acc) * 0.25
        out_f = _mm_big(h["founder"], w_r_mean) + b_l_mean
        for aggr_f, p in founder_acc:
            out_f = out_f + 0.25 * _mm_big(aggr_f, p["W_l"])
        new_h["founder"] = jax.nn.relu(out_f)
        h = new_h

    founder = h["founder"]
    cls = params["cls"]
    h1 = jax.nn.relu(_mm(founder, cls["W1"], bm=1024, bn=128, bk=128)
                     + cls["b1"])
    logits = _mm(h1, cls["W2"], bm=1024, bn=128, bk=128)[:, 0] + cls["b2"][0]
    return logits, founder


# speedup vs baseline: 1.2013x; 1.2013x over previous
import functools

import jax
import jax.numpy as jnp
from jax import lax
from jax.experimental import pallas as pl
from jax.experimental.pallas import tpu as pltpu
from jax.experimental.pallas import tpu_sc as plsc

_H = 128
_NF = 50000
_NU = 10000
_N_SMALL = {"worked_at": 10, "in": 150, "had": 50}

_E = 200000
_EP = 212992
_ROWS = _EP // 128
_RPT = _ROWS // 16
_G = 8
_NI = _RPT // _G
_NUP = 10112
_NFP = 50048


def _mm_kernel(x_ref, w_ref, o_ref, acc_ref, *, nk):
    @pl.when(pl.program_id(2) == 0)
    def _init():
        acc_ref[...] = jnp.zeros_like(acc_ref)

    acc_ref[...] += jnp.dot(x_ref[...], w_ref[...],
                            preferred_element_type=jnp.float32)

    @pl.when(pl.program_id(2) == nk - 1)
    def _fin():
        o_ref[...] = acc_ref[...]


def _ceil_to(x, m):
    return -(-x // m) * m


def _mm(x, w, bm, bn, bk):
    m, k = x.shape
    _, n = w.shape
    mp, kp, np_ = _ceil_to(m, bm), _ceil_to(k, bk), _ceil_to(n, bn)
    if mp > m or kp > k:
        x = jnp.pad(x, ((0, mp - m), (0, kp - k)))
    if kp > k or np_ > n:
        w = jnp.pad(w, ((0, kp - k), (0, np_ - n)))
    nk = kp // bk
    out = pl.pallas_call(
        functools.partial(_mm_kernel, nk=nk),
        grid=(mp // bm, np_ // bn, nk),
        in_specs=[
            pl.BlockSpec((bm, bk), lambda i, j, kk: (i, kk)),
            pl.BlockSpec((bk, bn), lambda i, j, kk: (kk, j)),
        ],
        out_specs=pl.BlockSpec((bm, bn), lambda i, j, kk: (i, j)),
        out_shape=jax.ShapeDtypeStruct((mp, np_), jnp.float32),
        scratch_shapes=[pltpu.VMEM((bm, bn), jnp.float32)],
        compiler_params=pltpu.CompilerParams(
            dimension_semantics=("parallel", "parallel", "arbitrary")),
    )(x, w)
    if mp > m or np_ > n:
        out = out[:m, :n]
    return out


def _mm_big(x, w):
    return _mm(x, w, bm=1024, bn=128, bk=_ceil_to(x.shape[1], 128))


def _sc_pass(s, gtab, gidx, sidx, out, n_out, acc, gbuf, sbuf, rows, sem,
             zeros_hbm):
    zr = n_out // 16
    pltpu.sync_copy(zeros_hbm.at[pl.ds(0, zr)], acc.at[pl.ds(s * zr, zr)])
    plsc.subcore_barrier()

    row0 = s * _RPT

    def body(j, carry):
        base = row0 + j * _G
        pltpu.sync_copy(gidx.at[pl.ds(base, _G)], gbuf)
        pltpu.sync_copy(sidx.at[pl.ds(base, _G)], sbuf)
        for half in range(2):
            cps = [pltpu.async_copy(
                gtab.at[gbuf.at[half * 4 + jj]], rows.at[jj], sem)
                for jj in range(4)]
            for cp in cps:
                cp.wait()
            for jj in range(4):
                pltpu.sync_copy(rows.at[jj], acc.at[sbuf.at[half * 4 + jj]],
                                add=True)
        return carry

    lax.fori_loop(0, _NI, body, 0)
    plsc.subcore_barrier()
    dr = n_out // 16
    pltpu.sync_copy(acc.at[pl.ds(s * dr, dr)], out.at[pl.ds(s * dr, dr)])
    plsc.subcore_barrier()


def _seg_body(hf0, hf1, hf2, hf3, hu0, hu1, hu2, hu3,
              gidx_f, sidx_f, gidx_r, sidx_r, zeros_hbm,
              ou0, ou1, ou2, ou3, of0, of1, of2, of3,
              acc, gbuf, sbuf, rows, sem):
    c = lax.axis_index("c")
    s = lax.axis_index("s")
    hf = (hf0, hf1, hf2, hf3)
    hu = (hu0, hu1, hu2, hu3)
    ou = (ou0, ou1, ou2, ou3)
    of = (of0, of1, of2, of3)
    for cv in (0, 1):
        @pl.when(c == cv)
        def _(cv=cv):
            for q in (2 * cv, 2 * cv + 1):
                _sc_pass(s, hf[q], gidx_f, sidx_f, ou[q], _NUP,
                         acc, gbuf, sbuf, rows, sem, zeros_hbm)
                _sc_pass(s, hu[q], gidx_r, sidx_r, of[q], _NFP,
                         acc, gbuf, sbuf, rows, sem, zeros_hbm)


def _make_seg_call():
    mesh = plsc.VectorSubcoreMesh(core_axis_name="c", subcore_axis_name="s")
    q_u = jax.ShapeDtypeStruct((_NUP, 32), jnp.float32)
    q_f = jax.ShapeDtypeStruct((_NFP, 32), jnp.float32)
    return pl.kernel(
        _seg_body,
        out_type=[q_u] * 4 + [q_f] * 4,
        mesh=mesh,
        scratch_types=[
            pltpu.VMEM_SHARED((_NFP, 32), jnp.float32),
            pltpu.VMEM((_G, 128), jnp.int32),
            pltpu.VMEM((_G, 128), jnp.int32),
            pltpu.VMEM((4, 128, 32), jnp.float32),
            pltpu.SemaphoreType.DMA,
        ],
        compiler_params=pltpu.CompilerParams(use_tc_tiling_on_sc=False),
    )


def _deg_pass(s, sidx, out, n_out, acc, sbuf, ones, sem, zeros_hbm):
    zr = n_out // 16
    pltpu.sync_copy(zeros_hbm.at[pl.ds(0, zr)], acc.at[pl.ds(s * zr, zr)])
    plsc.subcore_barrier()
    row0 = s * _RPT

    def body(j, carry):
        base = row0 + j * _G
        pltpu.sync_copy(sidx.at[pl.ds(base, _G)], sbuf)
        for jj in range(_G):
            pltpu.sync_copy(ones, acc.at[sbuf.at[jj]], add=True)
        return carry

    lax.fori_loop(0, _NI, body, 0)
    plsc.subcore_barrier()
    dr = n_out // 16
    pltpu.sync_copy(acc.at[pl.ds(s * dr, dr)], out.at[pl.ds(s * dr, dr)])


def _deg_body(sidx_f, sidx_r, zeros_hbm, ones_hbm, deg_u, deg_f,
              acc, sbuf, ones, sem):
    c = lax.axis_index("c")
    s = lax.axis_index("s")
    pltpu.sync_copy(ones_hbm, ones)

    @pl.when(c == 0)
    def _u():
        _deg_pass(s, sidx_f, deg_u, _NUP, acc, sbuf, ones, sem, zeros_hbm)

    @pl.when(c == 1)
    def _f():
        _deg_pass(s, sidx_r, deg_f, _NFP, acc, sbuf, ones, sem, zeros_hbm)


def _make_deg_call():
    mesh = plsc.VectorSubcoreMesh(core_axis_name="c", subcore_axis_name="s")
    return pl.kernel(
        _deg_body,
        out_type=[jax.ShapeDtypeStruct((_NUP, 16), jnp.float32),
                  jax.ShapeDtypeStruct((_NFP, 16), jnp.float32)],
        mesh=mesh,
        scratch_types=[
            pltpu.VMEM_SHARED((_NFP, 16), jnp.float32),
            pltpu.VMEM((_G, 128), jnp.int32),
            pltpu.VMEM((128, 16), jnp.float32),
            pltpu.SemaphoreType.DMA,
        ],
        compiler_params=pltpu.CompilerParams(use_tc_tiling_on_sc=False),
    )


def _pad_idx(idx, fill):
    return jnp.concatenate(
        [idx, jnp.full((_EP - _E,), fill, jnp.int32)]).reshape(_ROWS, 128)


def kernel(params, x_founder, x_university, x_company_size, x_industry,
           x_role_type, src_studied_at, dst_studied_at, src_worked_at,
           dst_worked_at, src_in, dst_in, src_had, dst_had):
    xs = {"founder": x_founder, "university": x_university,
          "company_size": x_company_size, "industry": x_industry,
          "role_type": x_role_type}
    rels = {"studied_at": ("university", src_studied_at, dst_studied_at),
            "worked_at": ("company_size", src_worked_at, dst_worked_at),
            "in": ("industry", src_in, dst_in),
            "had": ("role_type", src_had, dst_had)}

    proj = params["proj"]
    h = {}
    for nt, x in xs.items():
        bm = 1024 if x.shape[0] > 1024 else _ceil_to(x.shape[0], 8)
        h[nt] = _mm(x, proj[nt]["W"], bm=bm, bn=128,
                    bk=_ceil_to(x.shape[1], 128)) + proj[nt]["b"]

    M = {}
    for r, (nt, src, dst) in rels.items():
        if nt == "university":
            continue
        ns = _N_SMALL[r]
        M[r] = jnp.zeros((_NF, ns), jnp.float32).at[src, dst].add(1.0)

    gidx_f = _pad_idx(src_studied_at, 0)
    sidx_f = _pad_idx(dst_studied_at, _NU)
    gidx_r = _pad_idx(dst_studied_at, 0)
    sidx_r = _pad_idx(src_studied_at, _NF)
    zeros_hbm = jnp.zeros((3136, 32), jnp.float32)
    zeros16_hbm = jnp.zeros((3136, 16), jnp.float32)
    ones_hbm = jnp.ones((128, 16), jnp.float32)
    deg_call = _make_deg_call()
    deg_u16, deg_f16 = deg_call(sidx_f, sidx_r, zeros16_hbm, ones_hbm)
    deg_u = jnp.clip(deg_u16[:_NU, 0], 1.0, None)[:, None]
    deg_fu = jnp.clip(deg_f16[:_NF, 0], 1.0, None)[:, None]
    seg_call = _make_seg_call()

    ones_f = jnp.ones((_NF, 1), jnp.float32)

    for layer in params["convs"]:
        hfT_aug = jnp.concatenate(
            [h["founder"].T, jnp.ones((1, _NF), jnp.float32)], axis=0)
        new_h = {}
        founder_acc = []

        hfq = [h["founder"][:, q * 32:(q + 1) * 32] for q in range(4)]
        huq = [h["university"][:, q * 32:(q + 1) * 32] for q in range(4)]
        outs = seg_call(*hfq, *huq, gidx_f, sidx_f, gidx_r, sidx_r, zeros_hbm)
        sum_u = jnp.concatenate([o[:_NU] for o in outs[:4]], axis=1)
        rev_sum = jnp.concatenate([o[:_NF] for o in outs[4:]], axis=1)

        for r, (nt, src, dst) in rels.items():
            p_fwd = layer["founder__" + r + "__" + nt]
            p_rev = layer[nt + "__rev_" + r + "__founder"]
            if nt == "university":
                aggr_d = sum_u / deg_u
                aggr_f = rev_sum / deg_fu
            else:
                ns = _N_SMALL[r]
                s_t = _mm(hfT_aug, M[r], bm=136, bn=128, bk=1024)
                aggr_d = (s_t[:_H] / jnp.clip(s_t[_H:_H + 1], 1.0, None)).T
                h_s_aug = jnp.concatenate(
                    [h[nt], jnp.ones((ns, 1), jnp.float32)], axis=1)
                rsum = _mm(M[r], h_s_aug, bm=1024, bn=128,
                           bk=_ceil_to(ns + 1, 128))
                aggr_f = rsum[:, :_H] / jnp.clip(rsum[:, _H:_H + 1], 1.0,
                                                 None)

            bm_d = 1024 if aggr_d.shape[0] > 1024 else _ceil_to(
                aggr_d.shape[0], 8)
            out_d = (_mm(aggr_d, p_fwd["W_l"], bm=bm_d, bn=128, bk=128)
                     + p_fwd["b_l"]
                     + _mm(h[nt], p_fwd["W_r"], bm=bm_d, bn=128, bk=128))
            new_h[nt] = jax.nn.relu(out_d)
            founder_acc.append((aggr_f, p_rev))

        w_r_mean = sum(p["W_r"] for _, p in founder_acc) * 0.25
        b_l_mean = sum(p["b_l"] for _, p in founder_acc) * 0.25
        out_f = _mm_big(h["founder"], w_r_mean) + b_l_mean
        for aggr_f, p in founder_acc:
            out_f = out_f + 0.25 * _mm_big(aggr_f, p["W_l"])
        new_h["founder"] = jax.nn.relu(out_f)
        h = new_h

    founder = h["founder"]
    cls = params["cls"]
    h1 = jax.nn.relu(_mm(founder, cls["W1"], bm=1024, bn=128, bk=128)
                     + cls["b1"])
    logits = _mm(h1, cls["W2"], bm=1024, bn=128, bk=128)[:, 0] + cls["b2"][0]
    return logits, founder


# speedup vs baseline: 1.2172x; 1.0133x over previous
import functools

import jax
import jax.numpy as jnp
from jax import lax
from jax.experimental import pallas as pl
from jax.experimental.pallas import tpu as pltpu
from jax.experimental.pallas import tpu_sc as plsc

_H = 128
_NF = 50000
_NU = 10000
_N_SMALL = {"worked_at": 10, "in": 150, "had": 50}

_E = 200000
_EP = 212992
_ROWS = _EP // 128
_RPT = _ROWS // 16
_G = 8
_NI = _RPT // _G
_NUP = 10112
_NFP = 50048


def _mm_kernel(x_ref, w_ref, o_ref, acc_ref, *, nk):
    @pl.when(pl.program_id(2) == 0)
    def _init():
        acc_ref[...] = jnp.zeros_like(acc_ref)

    acc_ref[...] += jnp.dot(x_ref[...], w_ref[...],
                            preferred_element_type=jnp.float32)

    @pl.when(pl.program_id(2) == nk - 1)
    def _fin():
        o_ref[...] = acc_ref[...]


def _ceil_to(x, m):
    return -(-x // m) * m


def _mm(x, w, bm, bn, bk):
    m, k = x.shape
    _, n = w.shape
    mp, kp, np_ = _ceil_to(m, bm), _ceil_to(k, bk), _ceil_to(n, bn)
    if mp > m or kp > k:
        x = jnp.pad(x, ((0, mp - m), (0, kp - k)))
    if kp > k or np_ > n:
        w = jnp.pad(w, ((0, kp - k), (0, np_ - n)))
    nk = kp // bk
    out = pl.pallas_call(
        functools.partial(_mm_kernel, nk=nk),
        grid=(mp // bm, np_ // bn, nk),
        in_specs=[
            pl.BlockSpec((bm, bk), lambda i, j, kk: (i, kk)),
            pl.BlockSpec((bk, bn), lambda i, j, kk: (kk, j)),
        ],
        out_specs=pl.BlockSpec((bm, bn), lambda i, j, kk: (i, j)),
        out_shape=jax.ShapeDtypeStruct((mp, np_), jnp.float32),
        scratch_shapes=[pltpu.VMEM((bm, bn), jnp.float32)],
        compiler_params=pltpu.CompilerParams(
            dimension_semantics=("parallel", "parallel", "arbitrary")),
    )(x, w)
    if mp > m or np_ > n:
        out = out[:m, :n]
    return out


def _mm_big(x, w):
    return _mm(x, w, bm=1024, bn=128, bk=_ceil_to(x.shape[1], 128))


def _sc_pass(s, gtab, gidx, sidx, out, n_out, acc, gbuf, sbuf, rows,
             gsem, ssem, zeros_hbm):
    zr = n_out // 16
    pltpu.sync_copy(zeros_hbm.at[pl.ds(0, zr)], acc.at[pl.ds(s * zr, zr)])
    plsc.subcore_barrier()

    row0 = s * _RPT

    def body(j, carry):
        base = row0 + j * _G
        pltpu.sync_copy(gidx.at[pl.ds(base, _G)], gbuf)
        pltpu.sync_copy(sidx.at[pl.ds(base, _G)], sbuf)
        hs = []
        for g in range(4):
            b = g % 2
            if g >= 2:
                hs[2 * (g - 2)].wait()
                hs[2 * (g - 2) + 1].wait()
            gh = [pltpu.async_copy(gtab.at[gbuf.at[2 * g + t]],
                                   rows.at[b, t], gsem) for t in range(2)]
            for cp in gh:
                cp.wait()
            hs += [pltpu.async_copy(rows.at[b, t],
                                    acc.at[sbuf.at[2 * g + t]],
                                    ssem, add=True) for t in range(2)]
        for cp in hs[4:]:
            cp.wait()
        return carry

    lax.fori_loop(0, _NI, body, 0)
    plsc.subcore_barrier()
    dr = n_out // 16
    pltpu.sync_copy(acc.at[pl.ds(s * dr, dr)], out.at[pl.ds(s * dr, dr)])
    plsc.subcore_barrier()


def _seg_body(hf0, hf1, hf2, hf3, hu0, hu1, hu2, hu3,
              gidx_f, sidx_f, gidx_r, sidx_r, zeros_hbm,
              ou0, ou1, ou2, ou3, of0, of1, of2, of3,
              acc, gbuf, sbuf, rows, gsem, ssem):
    c = lax.axis_index("c")
    s = lax.axis_index("s")
    hf = (hf0, hf1, hf2, hf3)
    hu = (hu0, hu1, hu2, hu3)
    ou = (ou0, ou1, ou2, ou3)
    of = (of0, of1, of2, of3)
    for cv in (0, 1):
        @pl.when(c == cv)
        def _(cv=cv):
            for q in (2 * cv, 2 * cv + 1):
                _sc_pass(s, hf[q], gidx_f, sidx_f, ou[q], _NUP,
                         acc, gbuf, sbuf, rows, gsem, ssem, zeros_hbm)
                _sc_pass(s, hu[q], gidx_r, sidx_r, of[q], _NFP,
                         acc, gbuf, sbuf, rows, gsem, ssem, zeros_hbm)


def _make_seg_call():
    mesh = plsc.VectorSubcoreMesh(core_axis_name="c", subcore_axis_name="s")
    q_u = jax.ShapeDtypeStruct((_NUP, 32), jnp.float32)
    q_f = jax.ShapeDtypeStruct((_NFP, 32), jnp.float32)
    return pl.kernel(
        _seg_body,
        out_type=[q_u] * 4 + [q_f] * 4,
        mesh=mesh,
        scratch_types=[
            pltpu.VMEM_SHARED((_NFP, 32), jnp.float32),
            pltpu.VMEM((_G, 128), jnp.int32),
            pltpu.VMEM((_G, 128), jnp.int32),
            pltpu.VMEM((2, 2, 128, 32), jnp.float32),
            pltpu.SemaphoreType.DMA,
            pltpu.SemaphoreType.DMA,
        ],
        compiler_params=pltpu.CompilerParams(use_tc_tiling_on_sc=False),
    )


def _deg_pass(s, sidx, out, n_out, acc, sbuf, ones, sem, zeros_hbm):
    zr = n_out // 16
    pltpu.sync_copy(zeros_hbm.at[pl.ds(0, zr)], acc.at[pl.ds(s * zr, zr)])
    plsc.subcore_barrier()
    row0 = s * _RPT

    def body(j, carry):
        base = row0 + j * _G
        pltpu.sync_copy(sidx.at[pl.ds(base, _G)], sbuf)
        for jj in range(_G):
            pltpu.sync_copy(ones, acc.at[sbuf.at[jj]], add=True)
        return carry

    lax.fori_loop(0, _NI, body, 0)
    plsc.subcore_barrier()
    dr = n_out // 16
    pltpu.sync_copy(acc.at[pl.ds(s * dr, dr)], out.at[pl.ds(s * dr, dr)])


def _deg_body(sidx_f, sidx_r, zeros_hbm, ones_hbm, deg_u, deg_f,
              acc, sbuf, ones, sem):
    c = lax.axis_index("c")
    s = lax.axis_index("s")
    pltpu.sync_copy(ones_hbm, ones)

    @pl.when(c == 0)
    def _u():
        _deg_pass(s, sidx_f, deg_u, _NUP, acc, sbuf, ones, sem, zeros_hbm)

    @pl.when(c == 1)
    def _f():
        _deg_pass(s, sidx_r, deg_f, _NFP, acc, sbuf, ones, sem, zeros_hbm)


def _make_deg_call():
    mesh = plsc.VectorSubcoreMesh(core_axis_name="c", subcore_axis_name="s")
    return pl.kernel(
        _deg_body,
        out_type=[jax.ShapeDtypeStruct((_NUP, 16), jnp.float32),
                  jax.ShapeDtypeStruct((_NFP, 16), jnp.float32)],
        mesh=mesh,
        scratch_types=[
            pltpu.VMEM_SHARED((_NFP, 16), jnp.float32),
            pltpu.VMEM((_G, 128), jnp.int32),
            pltpu.VMEM((128, 16), jnp.float32),
            pltpu.SemaphoreType.DMA,
        ],
        compiler_params=pltpu.CompilerParams(use_tc_tiling_on_sc=False),
    )


def _pad_idx(idx, fill):
    return jnp.concatenate(
        [idx, jnp.full((_EP - _E,), fill, jnp.int32)]).reshape(_ROWS, 128)


def kernel(params, x_founder, x_university, x_company_size, x_industry,
           x_role_type, src_studied_at, dst_studied_at, src_worked_at,
           dst_worked_at, src_in, dst_in, src_had, dst_had):
    xs = {"founder": x_founder, "university": x_university,
          "company_size": x_company_size, "industry": x_industry,
          "role_type": x_role_type}
    rels = {"studied_at": ("university", src_studied_at, dst_studied_at),
            "worked_at": ("company_size", src_worked_at, dst_worked_at),
            "in": ("industry", src_in, dst_in),
            "had": ("role_type", src_had, dst_had)}

    proj = params["proj"]
    h = {}
    for nt, x in xs.items():
        bm = 1024 if x.shape[0] > 1024 else _ceil_to(x.shape[0], 8)
        h[nt] = _mm(x, proj[nt]["W"], bm=bm, bn=128,
                    bk=_ceil_to(x.shape[1], 128)) + proj[nt]["b"]

    M = {}
    for r, (nt, src, dst) in rels.items():
        if nt == "university":
            continue
        ns = _N_SMALL[r]
        M[r] = jnp.zeros((_NF, ns), jnp.float32).at[src, dst].add(1.0)

    gidx_f = _pad_idx(src_studied_at, 0)
    sidx_f = _pad_idx(dst_studied_at, _NU)
    gidx_r = _pad_idx(dst_studied_at, 0)
    sidx_r = _pad_idx(src_studied_at, _NF)
    zeros_hbm = jnp.zeros((3136, 32), jnp.float32)
    zeros16_hbm = jnp.zeros((3136, 16), jnp.float32)
    ones_hbm = jnp.ones((128, 16), jnp.float32)
    deg_call = _make_deg_call()
    deg_u16, deg_f16 = deg_call(sidx_f, sidx_r, zeros16_hbm, ones_hbm)
    deg_u = jnp.clip(deg_u16[:_NU, 0], 1.0, None)[:, None]
    deg_fu = jnp.clip(deg_f16[:_NF, 0], 1.0, None)[:, None]
    seg_call = _make_seg_call()

    ones_f = jnp.ones((_NF, 1), jnp.float32)

    for layer in params["convs"]:
        hfT_aug = jnp.concatenate(
            [h["founder"].T, jnp.ones((1, _NF), jnp.float32)], axis=0)
        new_h = {}
        founder_acc = []

        hfq = [h["founder"][:, q * 32:(q + 1) * 32] for q in range(4)]
        huq = [h["university"][:, q * 32:(q + 1) * 32] for q in range(4)]
        outs = seg_call(*hfq, *huq, gidx_f, sidx_f, gidx_r, sidx_r, zeros_hbm)
        sum_u = jnp.concatenate([o[:_NU] for o in outs[:4]], axis=1)
        rev_sum = jnp.concatenate([o[:_NF] for o in outs[4:]], axis=1)

        for r, (nt, src, dst) in rels.items():
            p_fwd = layer["founder__" + r + "__" + nt]
            p_rev = layer[nt + "__rev_" + r + "__founder"]
            if nt == "university":
                aggr_d = sum_u / deg_u
                aggr_f = rev_sum / deg_fu
            else:
                ns = _N_SMALL[r]
                s_t = _mm(hfT_aug, M[r], bm=136, bn=128, bk=1024)
                aggr_d = (s_t[:_H] / jnp.clip(s_t[_H:_H + 1], 1.0, None)).T
                h_s_aug = jnp.concatenate(
                    [h[nt], jnp.ones((ns, 1), jnp.float32)], axis=1)
                rsum = _mm(M[r], h_s_aug, bm=1024, bn=128,
                           bk=_ceil_to(ns + 1, 128))
                aggr_f = rsum[:, :_H] / jnp.clip(rsum[:, _H:_H + 1], 1.0,
                                                 None)

            bm_d = 1024 if aggr_d.shape[0] > 1024 else _ceil_to(
                aggr_d.shape[0], 8)
            out_d = (_mm(aggr_d, p_fwd["W_l"], bm=bm_d, bn=128, bk=128)
                     + p_fwd["b_l"]
                     + _mm(h[nt], p_fwd["W_r"], bm=bm_d, bn=128, bk=128))
            new_h[nt] = jax.nn.relu(out_d)
            founder_acc.append((aggr_f, p_rev))

        w_r_mean = sum(p["W_r"] for _, p in founder_acc) * 0.25
        b_l_mean = sum(p["b_l"] for _, p in founder_acc) * 0.25
        out_f = _mm_big(h["founder"], w_r_mean) + b_l_mean
        for aggr_f, p in founder_acc:
            out_f = out_f + 0.25 * _mm_big(aggr_f, p["W_l"])
        new_h["founder"] = jax.nn.relu(out_f)
        h = new_h

    founder = h["founder"]
    cls = params["cls"]
    h1 = jax.nn.relu(_mm(founder, cls["W1"], bm=1024, bn=128, bk=128)
                     + cls["b1"])
    logits = _mm(h1, cls["W2"], bm=1024, bn=128, bk=128)[:, 0] + cls["b2"][0]
    return logits, founder
